# baseline (speedup 1.0000x reference)
import jax
import jax.numpy as jnp
from jax.experimental import pallas as pl
from jax.experimental.pallas import tpu as pltpu

B = 16384
A = 6
AP = 128
H = 256
D = 64
K = 1024
BETA = 0.25
RECONS_W = 1.0

BBLK = 1024
NBLK = B // BBLK


def _enc_kernel(a_ref, w1_ref, b1_ref, w2_ref, b2_ref, w3_ref, b3_ref, e_ref):
    h = jnp.maximum(jnp.dot(a_ref[...], w1_ref[...],
                            preferred_element_type=jnp.float32) + b1_ref[...], 0.0)
    h = jnp.maximum(jnp.dot(h, w2_ref[...],
                            preferred_element_type=jnp.float32) + b2_ref[...], 0.0)
    e_ref[...] = jnp.dot(h, w3_ref[...],
                         preferred_element_type=jnp.float32) + b3_ref[...]


def _vq_dec_kernel(a_ref, e_ref, en_ref, cb_ref, cbt_ref,
                   wd1_ref, bd1_ref, wd2_ref, bd2_ref, wd3_ref, bd3_ref,
                   idx_ref, q_ref, acc_ref):
    i = pl.program_id(0)
    e = e_ref[...]
    cbt = cbt_ref[...]
    cn = jnp.sum(cbt * cbt, axis=0, keepdims=True)
    d2 = (en_ref[...] - 2.0 * jnp.dot(e, cbt, preferred_element_type=jnp.float32)) + cn
    dist = jnp.sqrt(jnp.maximum(d2, 0.0))
    m = jnp.min(dist, axis=1, keepdims=True)
    lanes = jax.lax.broadcasted_iota(jnp.int32, (BBLK, K), 1)
    idx = jnp.min(jnp.where(dist == m, lanes, jnp.int32(K)), axis=1).astype(jnp.int32)
    idx_ref[...] = idx[None, None, :]
    onehot = (idx[:, None] == lanes).astype(jnp.float32)
    q = jnp.dot(onehot, cb_ref[...], preferred_element_type=jnp.float32)
    qst = e + (q - e)
    q_ref[...] = qst
    hd = jnp.maximum(jnp.dot(qst, wd1_ref[...],
                             preferred_element_type=jnp.float32) + bd1_ref[...], 0.0)
    hd = jnp.maximum(jnp.dot(hd, wd2_ref[...],
                             preferred_element_type=jnp.float32) + bd2_ref[...], 0.0)
    r = jnp.tanh(jnp.dot(hd, wd3_ref[...],
                         preferred_element_type=jnp.float32) + bd3_ref[...])
    dq = q - e
    dr = r - a_ref[...]
    lane = jax.lax.broadcasted_iota(jnp.int32, (1, 128), 1)
    vals = jnp.where(lane == 0, jnp.sum(dq * dq),
                     jnp.where(lane == 1, jnp.sum(dr * dr), 0.0))

    @pl.when(i == 0)
    def _init():
        acc_ref[...] = vals

    @pl.when(i > 0)
    def _accum():
        acc_ref[...] = acc_ref[...] + vals


def kernel(action, W1, b1, W2, b2, W3, b3, codebook,
           Wd1, bd1, Wd2, bd2, Wd3, bd3):
    f32 = jnp.float32
    a_p = jnp.zeros((B, AP), f32).at[:, :A].set(action)
    w1_p = jnp.zeros((AP, H), f32).at[:A, :].set(W1)
    wd3_p = jnp.zeros((H, AP), f32).at[:, :A].set(Wd3)
    bd3_p = jnp.zeros((1, AP), f32).at[0, :A].set(bd3)
    cbt = codebook.T

    rep = lambda i: (0, 0)
    e = pl.pallas_call(
        _enc_kernel,
        grid=(NBLK,),
        in_specs=[
            pl.BlockSpec((BBLK, AP), lambda i: (i, 0)),
            pl.BlockSpec((AP, H), rep),
            pl.BlockSpec((1, H), rep),
            pl.BlockSpec((H, H), rep),
            pl.BlockSpec((1, H), rep),
            pl.BlockSpec((H, D), rep),
            pl.BlockSpec((1, D), rep),
        ],
        out_specs=pl.BlockSpec((BBLK, D), lambda i: (i, 0)),
        out_shape=jax.ShapeDtypeStruct((B, D), f32),
        compiler_params=pltpu.CompilerParams(
            dimension_semantics=("arbitrary",)),
    )(a_p, w1_p, b1.reshape(1, H), W2, b2.reshape(1, H), W3, b3.reshape(1, D))

    en = jnp.sum(e ** 2, axis=1, keepdims=True)

    idx_out, q_out, acc = pl.pallas_call(
        _vq_dec_kernel,
        grid=(NBLK,),
        in_specs=[
            pl.BlockSpec((BBLK, AP), lambda i: (i, 0)),
            pl.BlockSpec((BBLK, D), lambda i: (i, 0)),
            pl.BlockSpec((BBLK, 1), lambda i: (i, 0)),
            pl.BlockSpec((K, D), rep),
            pl.BlockSpec((D, K), rep),
            pl.BlockSpec((D, H), rep),
            pl.BlockSpec((1, H), rep),
            pl.BlockSpec((H, H), rep),
            pl.BlockSpec((1, H), rep),
            pl.BlockSpec((H, AP), rep),
            pl.BlockSpec((1, AP), rep),
        ],
        out_specs=[
            pl.BlockSpec((1, 1, BBLK), lambda i: (i, 0, 0)),
            pl.BlockSpec((BBLK, D), lambda i: (i, 0)),
            pl.BlockSpec((1, 128), rep),
        ],
        out_shape=[
            jax.ShapeDtypeStruct((NBLK, 1, BBLK), jnp.int32),
            jax.ShapeDtypeStruct((B, D), f32),
            jax.ShapeDtypeStruct((1, 128), f32),
        ],
        compiler_params=pltpu.CompilerParams(
            dimension_semantics=("arbitrary",)),
    )(a_p, e, en, codebook, cbt, Wd1, bd1.reshape(1, H), Wd2,
      bd2.reshape(1, H), wd3_p, bd3_p)

    quantized_index = idx_out.reshape(B)
    quantized_embedding = q_out
    vq_mse = acc[0, 0] / (B * D)
    commitment_loss = vq_mse
    embedding_loss = vq_mse
    vq_loss = commitment_loss * BETA + embedding_loss
    recons_loss = acc[0, 1] / (B * A)
    total = RECONS_W * recons_loss + vq_loss
    return (total, recons_loss, vq_loss, embedding_loss, commitment_loss,
            quantized_index, quantized_embedding)
